# Initial kernel scaffold; baseline (speedup 1.0000x reference)
import jax
import jax.numpy as jnp
from jax.experimental import pallas as pl


def kernel(node_features, edge_attr, senders, receivers, params):
    raise NotImplementedError("write your pallas kernel here")


# speedup vs baseline: 3.5183x; 3.5183x over previous
import functools

import jax
import jax.numpy as jnp
from jax import lax
from jax.experimental import pallas as pl
from jax.experimental.pallas import tpu as pltpu
from jax.experimental.pallas import tpu_sc as plsc

H = 128
NC = 2
NS = 16
NW = NC * NS
CH = 128
BE = 8000

_F32 = jnp.float32


def _ln(y, scale, bias):
    mu = jnp.mean(y, axis=-1, keepdims=True)
    yc = y - mu
    var = jnp.mean(yc * yc, axis=-1, keepdims=True)
    return yc * lax.rsqrt(var + 1e-5) * scale + bias


def _dot(a, b):
    return jnp.dot(a, b, preferred_element_type=_F32)


def _node_enc_body(x_ref, w1, b1, w2, b2, ls, lb, w1s, w1r, xo, xso, xro):
    h = jnp.maximum(_dot(x_ref[:], w1[:]) + b1[:], 0.0)
    x = _ln(_dot(h, w2[:]) + b2[:], ls[:], lb[:])
    xo[:] = x
    xso[:] = _dot(x, w1s[:])
    xro[:] = _dot(x, w1r[:])


def _edge_enc_body(a_ref, w1, b1, w2, b2, ls, lb, eo):
    h = jnp.maximum(_dot(a_ref[:], w1[:]) + b1[:], 0.0)
    eo[:] = _ln(_dot(h, w2[:]) + b2[:], ls[:], lb[:])


def _edge_upd_body(e_ref, g_ref, w1e, b1, w2, b2, ls, lb, eo):
    h = jnp.maximum(_dot(e_ref[:], w1e[:]) + g_ref[:] + b1[:], 0.0)
    eo[:] = _ln(_dot(h, w2[:]) + b2[:], ls[:], lb[:]) + e_ref[:]


def _node_upd_body(x_ref, agg_ref, w1x, w1a, b1, w2, b2, ls, lb, w1s, w1r,
                   xo, xso, xro):
    agg = agg_ref[0] + agg_ref[1]
    h = jnp.maximum(_dot(x_ref[:], w1x[:]) + _dot(agg, w1a[:]) + b1[:], 0.0)
    x = _ln(_dot(h, w2[:]) + b2[:], ls[:], lb[:]) + x_ref[:]
    xo[:] = x
    xso[:] = _dot(x, w1s[:])
    xro[:] = _dot(x, w1r[:])


def _node_dec_body(x_ref, agg_ref, w1x, w1a, b1, w2, b2, ls, lb,
                   wd1, bd1, wd2, bd2, oo):
    agg = agg_ref[0] + agg_ref[1]
    h = jnp.maximum(_dot(x_ref[:], w1x[:]) + _dot(agg, w1a[:]) + b1[:], 0.0)
    x = _ln(_dot(h, w2[:]) + b2[:], ls[:], lb[:]) + x_ref[:]
    hd = jnp.maximum(_dot(x, wd1[:]) + bd1[:], 0.0)
    oo[:] = _dot(hd, wd2[:]) + bd2[:]


def _row(v):
    return v.reshape(1, -1)


@functools.lru_cache(maxsize=None)
def _make_gather(n, e):
    nchunk = e // CH
    mesh = plsc.VectorSubcoreMesh(core_axis_name="c", subcore_axis_name="s")

    @functools.partial(
        pl.kernel,
        out_type=jax.ShapeDtypeStruct((e, H), _F32),
        mesh=mesh,
        scratch_types=[
            pltpu.VMEM((CH,), jnp.int32),
            pltpu.VMEM((CH,), jnp.int32),
            pltpu.VMEM((CH, H), _F32),
            pltpu.VMEM((CH, H), _F32),
            pltpu.SemaphoreType.DMA,
            pltpu.SemaphoreType.DMA,
        ],
    )
    def gather(xs_hbm, xr_hbm, s_hbm, r_hbm, out_hbm,
               sidx, ridx, bufa, bufb, sema, semb):
        wid = lax.axis_index("s") * NC + lax.axis_index("c")
        nloc = (nchunk - wid + NW - 1) // NW

        def body(i, carry):
            base = (wid + i * NW) * CH
            pltpu.sync_copy(s_hbm.at[pl.ds(base, CH)], sidx)
            pltpu.sync_copy(r_hbm.at[pl.ds(base, CH)], ridx)
            cpa = pltpu.async_copy(xs_hbm.at[sidx], bufa, sema)
            cpb = pltpu.async_copy(xr_hbm.at[ridx], bufb, semb)
            cpa.wait()
            cpb.wait()

            def addrow(rr, c2):
                for j in range(H // 16):
                    sl = pl.ds(j * 16, 16)
                    bufa[rr, sl] = bufa[rr, sl] + bufb[rr, sl]
                return c2

            lax.fori_loop(0, CH, addrow, 0)
            pltpu.sync_copy(bufa, out_hbm.at[pl.ds(base, CH)])
            return carry

        lax.fori_loop(0, nloc, body, 0)

    return gather


@functools.lru_cache(maxsize=None)
def _make_scatter(n, e):
    nchunk = e // CH
    stripe = (n // NS) // 8 * 8
    tail = n - stripe * NS
    zrows = 104
    nz = stripe // zrows
    mesh = plsc.VectorSubcoreMesh(core_axis_name="c", subcore_axis_name="s")

    @functools.partial(
        pl.kernel,
        out_type=jax.ShapeDtypeStruct((NC, n, H), _F32),
        mesh=mesh,
        scratch_types=[
            pltpu.VMEM((CH,), jnp.int32),
            pltpu.VMEM((CH, H), _F32),
            pltpu.VMEM((zrows, H), _F32),
            pltpu.VMEM_SHARED((n, H), _F32),
        ],
    )
    def scatter(vals_hbm, r_hbm, out_hbm, idxv, rowsv, zbuf, acc):
        cid = lax.axis_index("c")
        sid = lax.axis_index("s")
        wid = sid * NC + cid

        def zrow(i, c2):
            for j in range(H // 16):
                zbuf[i, pl.ds(j * 16, 16)] = jnp.zeros((16,), _F32)
            return c2

        lax.fori_loop(0, zrows, zrow, 0)

        def zcp(k, c2):
            pltpu.sync_copy(zbuf, acc.at[pl.ds(sid * stripe + k * zrows, zrows)])
            return c2

        lax.fori_loop(0, nz, zcp, 0)
        if tail:
            @pl.when(sid == NS - 1)
            def _():
                pltpu.sync_copy(zbuf.at[pl.ds(0, tail)],
                                acc.at[pl.ds(NS * stripe, tail)])
        plsc.subcore_barrier()

        nloc = (nchunk - wid + NW - 1) // NW

        def body(i, carry):
            base = (wid + i * NW) * CH
            pltpu.sync_copy(r_hbm.at[pl.ds(base, CH)], idxv)
            pltpu.sync_copy(vals_hbm.at[pl.ds(base, CH)], rowsv)
            pltpu.sync_copy(rowsv, acc.at[idxv], add=True)
            return carry

        lax.fori_loop(0, nloc, body, 0)
        plsc.subcore_barrier()
        pltpu.sync_copy(
            acc.at[pl.ds(sid * stripe, stripe)],
            out_hbm.at[cid, pl.ds(sid * stripe, stripe)])
        if tail:
            @pl.when(sid == NS - 1)
            def _():
                pltpu.sync_copy(
                    acc.at[pl.ds(NS * stripe, tail)],
                    out_hbm.at[cid, pl.ds(NS * stripe, tail)])

    return scatter


def _w(p):
    return (p["W1"], _row(p["b1"]), p["W2"], _row(p["b2"]),
            _row(p["ln_scale"]), _row(p["ln_bias"]))


def kernel(node_features, edge_attr, senders, receivers, params):
    n = node_features.shape[0]
    e = edge_attr.shape[0]
    blocks = params["blocks"]
    nblk = len(blocks)

    hh = ((H, H), lambda i: (0, 0))

    def wspec(shape):
        return pl.BlockSpec(shape, lambda i: (0, 0))

    w1_0 = blocks[0]["edge_mlp"]["W1"]
    x, xs, xr = pl.pallas_call(
        _node_enc_body,
        out_shape=[jax.ShapeDtypeStruct((n, H), _F32)] * 3,
    )(node_features, *_w(params["node_enc"]),
      w1_0[H:2 * H], w1_0[2 * H:3 * H])

    grid = (e // BE,)
    ein = edge_attr.shape[1]
    we = _w(params["edge_enc"])
    ee = pl.pallas_call(
        _edge_enc_body,
        grid=grid,
        in_specs=[
            pl.BlockSpec((BE, ein), lambda i: (i, 0)),
            wspec((ein, H)), wspec((1, H)), wspec((H, H)), wspec((1, H)),
            wspec((1, H)), wspec((1, H)),
        ],
        out_specs=pl.BlockSpec((BE, H), lambda i: (i, 0)),
        out_shape=jax.ShapeDtypeStruct((e, H), _F32),
    )(edge_attr, *we)

    gather = _make_gather(n, e)
    scatter = _make_scatter(n, e)

    def edge_update(ecur, g, emlp):
        w1, b1, w2, b2, ls, lb = _w(emlp)
        return pl.pallas_call(
            _edge_upd_body,
            grid=grid,
            in_specs=[
                pl.BlockSpec((BE, H), lambda i: (i, 0)),
                pl.BlockSpec((BE, H), lambda i: (i, 0)),
                wspec((H, H)), wspec((1, H)), wspec((H, H)), wspec((1, H)),
                wspec((1, H)), wspec((1, H)),
            ],
            out_specs=pl.BlockSpec((BE, H), lambda i: (i, 0)),
            out_shape=jax.ShapeDtypeStruct((e, H), _F32),
        )(ecur, g, w1[:H], b1, w2, b2, ls, lb)

    ecur = ee
    for i, blk in enumerate(blocks):
        g = gather(xs, xr, senders, receivers)
        ecur = edge_update(ecur, g, blk["edge_mlp"])
        agg2 = scatter(ecur, receivers)
        nm = blk["node_mlp"]
        w1, b1, w2, b2, ls, lb = _w(nm)
        if i + 1 < nblk:
            w1n = blocks[i + 1]["edge_mlp"]["W1"]
            x, xs, xr = pl.pallas_call(
                _node_upd_body,
                out_shape=[jax.ShapeDtypeStruct((n, H), _F32)] * 3,
            )(x, agg2, w1[:H], w1[H:2 * H], b1, w2, b2, ls, lb,
              w1n[H:2 * H], w1n[2 * H:3 * H])
        else:
            dec = params["decoder"]
            out = pl.pallas_call(
                _node_dec_body,
                out_shape=jax.ShapeDtypeStruct((n, dec["W2"].shape[1]), _F32),
            )(x, agg2, w1[:H], w1[H:2 * H], b1, w2, b2, ls, lb,
              dec["W1"], _row(dec["b1"]), dec["W2"], _row(dec["b2"]))
    return out


# speedup vs baseline: 4.9934x; 1.4192x over previous
import functools

import jax
import jax.numpy as jnp
from jax import lax
from jax.experimental import pallas as pl
from jax.experimental.pallas import tpu as pltpu
from jax.experimental.pallas import tpu_sc as plsc

H = 128
NC = 2
NS = 16
NW = NC * NS
CH = 128
BE = 8000

_F32 = jnp.float32


def _ln(y, scale, bias):
    mu = jnp.mean(y, axis=-1, keepdims=True)
    yc = y - mu
    var = jnp.mean(yc * yc, axis=-1, keepdims=True)
    return yc * lax.rsqrt(var + 1e-5) * scale + bias


def _dot(a, b):
    return jnp.dot(a, b, preferred_element_type=_F32)


def _node_enc_body(x_ref, w1, b1, w2, b2, ls, lb, w1s, w1r, xo, xso, xro):
    h = jnp.maximum(_dot(x_ref[:], w1[:]) + b1[:], 0.0)
    x = _ln(_dot(h, w2[:]) + b2[:], ls[:], lb[:])
    xo[:] = x
    xso[:] = _dot(x, w1s[:])
    xro[:] = _dot(x, w1r[:])


def _edge_enc_body(a_ref, w1, b1, w2, b2, ls, lb, eo):
    h = jnp.maximum(_dot(a_ref[:], w1[:]) + b1[:], 0.0)
    eo[:] = _ln(_dot(h, w2[:]) + b2[:], ls[:], lb[:])


def _edge_upd_body(e_ref, g_ref, w1e, b1, w2, b2, ls, lb, eo):
    h = jnp.maximum(_dot(e_ref[:], w1e[:]) + g_ref[:] + b1[:], 0.0)
    eo[:] = _ln(_dot(h, w2[:]) + b2[:], ls[:], lb[:]) + e_ref[:]


def _node_upd_body(x_ref, agg_ref, w1x, w1a, b1, w2, b2, ls, lb, w1s, w1r,
                   xo, xso, xro):
    agg = agg_ref[0] + agg_ref[1]
    h = jnp.maximum(_dot(x_ref[:], w1x[:]) + _dot(agg, w1a[:]) + b1[:], 0.0)
    x = _ln(_dot(h, w2[:]) + b2[:], ls[:], lb[:]) + x_ref[:]
    xo[:] = x
    xso[:] = _dot(x, w1s[:])
    xro[:] = _dot(x, w1r[:])


def _node_dec_body(x_ref, agg_ref, w1x, w1a, b1, w2, b2, ls, lb,
                   wd1, bd1, wd2, bd2, oo):
    agg = agg_ref[0] + agg_ref[1]
    h = jnp.maximum(_dot(x_ref[:], w1x[:]) + _dot(agg, w1a[:]) + b1[:], 0.0)
    x = _ln(_dot(h, w2[:]) + b2[:], ls[:], lb[:]) + x_ref[:]
    hd = jnp.maximum(_dot(x, wd1[:]) + bd1[:], 0.0)
    oo[:] = _dot(hd, wd2[:]) + bd2[:]


def _row(v):
    return v.reshape(1, -1)


@functools.lru_cache(maxsize=None)
def _make_gather(n, e):
    nchunk = e // CH
    mesh = plsc.VectorSubcoreMesh(core_axis_name="c", subcore_axis_name="s")

    @functools.partial(
        pl.kernel,
        out_type=jax.ShapeDtypeStruct((e, H), _F32),
        mesh=mesh,
        scratch_types=[
            pltpu.VMEM((2, CH), jnp.int32),
            pltpu.VMEM((2, CH), jnp.int32),
            pltpu.VMEM((2, CH, H), _F32),
            pltpu.VMEM((2, CH, H), _F32),
            pltpu.SemaphoreType.DMA,
            pltpu.SemaphoreType.DMA,
            pltpu.SemaphoreType.DMA,
            pltpu.SemaphoreType.DMA,
        ],
    )
    def gather(xs_hbm, xr_hbm, s_hbm, r_hbm, out_hbm,
               sidx, ridx, bufa, bufb, semg0, semg1, semw0, semw1):
        wid = lax.axis_index("s") * NC + lax.axis_index("c")
        nloc = (nchunk - wid + NW - 1) // NW
        semg = (semg0, semg1)
        semw = (semw0, semw1)

        def fetch(k, b):
            base = (wid + k * NW) * CH
            pltpu.sync_copy(s_hbm.at[pl.ds(base, CH)], sidx.at[b])
            pltpu.sync_copy(r_hbm.at[pl.ds(base, CH)], ridx.at[b])
            pltpu.async_copy(xs_hbm.at[sidx.at[b]], bufa.at[b], semg[b])
            pltpu.async_copy(xr_hbm.at[ridx.at[b]], bufb.at[b], semg[b])

        fetch(0, 0)

        def group(g, carry):
            for b in range(2):
                k = 2 * g + b

                @pl.when(k < nloc)
                def _():
                    nb = 1 - b

                    @pl.when(k + 1 < nloc)
                    def _():
                        @pl.when(k >= 1)
                        def _():
                            pltpu.make_async_copy(
                                bufa.at[nb], out_hbm.at[pl.ds(0, CH)],
                                semw[nb]).wait()

                        fetch(k + 1, nb)

                    pltpu.make_async_copy(
                        xs_hbm.at[sidx.at[b]], bufa.at[b], semg[b]).wait()
                    pltpu.make_async_copy(
                        xr_hbm.at[ridx.at[b]], bufb.at[b], semg[b]).wait()

                    def addrow(rr, c2):
                        for j in range(H // 16):
                            sl = pl.ds(j * 16, 16)
                            bufa[b, rr, sl] = bufa[b, rr, sl] + bufb[b, rr, sl]
                        return c2

                    lax.fori_loop(0, CH, addrow, 0)
                    base = (wid + k * NW) * CH
                    pltpu.async_copy(bufa.at[b], out_hbm.at[pl.ds(base, CH)],
                                     semw[b])
            return carry

        lax.fori_loop(0, (nloc + 1) // 2, group, 0)
        for b in range(2):
            pltpu.make_async_copy(
                bufa.at[b], out_hbm.at[pl.ds(0, CH)], semw[b]).wait()

    return gather


@functools.lru_cache(maxsize=None)
def _make_scatter(n, e):
    nchunk = e // CH
    stripe = (n // NS) // 8 * 8
    tail = n - stripe * NS
    zrows = 104
    nz = stripe // zrows
    mesh = plsc.VectorSubcoreMesh(core_axis_name="c", subcore_axis_name="s")

    @functools.partial(
        pl.kernel,
        out_type=jax.ShapeDtypeStruct((NC, n, H), _F32),
        mesh=mesh,
        scratch_types=[
            pltpu.VMEM((2, CH), jnp.int32),
            pltpu.VMEM((2, CH, H), _F32),
            pltpu.VMEM((zrows, H), _F32),
            pltpu.VMEM_SHARED((n, H), _F32),
            pltpu.SemaphoreType.DMA,
            pltpu.SemaphoreType.DMA,
        ],
    )
    def scatter(vals_hbm, r_hbm, out_hbm, idxv, rowsv, zbuf, acc,
                semr0, semr1):
        cid = lax.axis_index("c")
        sid = lax.axis_index("s")
        wid = sid * NC + cid
        semr = (semr0, semr1)

        def zrow(i, c2):
            for j in range(H // 16):
                zbuf[i, pl.ds(j * 16, 16)] = jnp.zeros((16,), _F32)
            return c2

        lax.fori_loop(0, zrows, zrow, 0)

        def zcp(k, c2):
            pltpu.sync_copy(zbuf, acc.at[pl.ds(sid * stripe + k * zrows, zrows)])
            return c2

        lax.fori_loop(0, nz, zcp, 0)
        if tail:
            @pl.when(sid == NS - 1)
            def _():
                pltpu.sync_copy(zbuf.at[pl.ds(0, tail)],
                                acc.at[pl.ds(NS * stripe, tail)])
        plsc.subcore_barrier()

        nloc = (nchunk - wid + NW - 1) // NW

        def fetch(k, b):
            base = (wid + k * NW) * CH
            pltpu.async_copy(r_hbm.at[pl.ds(base, CH)], idxv.at[b], semr[b])
            pltpu.async_copy(vals_hbm.at[pl.ds(base, CH)], rowsv.at[b],
                             semr[b])

        fetch(0, 0)

        def group(g, carry):
            for b in range(2):
                k = 2 * g + b

                @pl.when(k < nloc)
                def _():
                    @pl.when(k + 1 < nloc)
                    def _():
                        fetch(k + 1, 1 - b)

                    pltpu.make_async_copy(
                        r_hbm.at[pl.ds(0, CH)], idxv.at[b], semr[b]).wait()
                    pltpu.make_async_copy(
                        vals_hbm.at[pl.ds(0, CH)], rowsv.at[b],
                        semr[b]).wait()
                    pltpu.sync_copy(rowsv.at[b], acc.at[idxv.at[b]], add=True)
            return carry

        lax.fori_loop(0, (nloc + 1) // 2, group, 0)
        plsc.subcore_barrier()
        pltpu.sync_copy(
            acc.at[pl.ds(sid * stripe, stripe)],
            out_hbm.at[cid, pl.ds(sid * stripe, stripe)])
        if tail:
            @pl.when(sid == NS - 1)
            def _():
                pltpu.sync_copy(
                    acc.at[pl.ds(NS * stripe, tail)],
                    out_hbm.at[cid, pl.ds(NS * stripe, tail)])

    return scatter


def _w(p):
    return (p["W1"], _row(p["b1"]), p["W2"], _row(p["b2"]),
            _row(p["ln_scale"]), _row(p["ln_bias"]))


def kernel(node_features, edge_attr, senders, receivers, params):
    n = node_features.shape[0]
    e = edge_attr.shape[0]
    blocks = params["blocks"]
    nblk = len(blocks)

    hh = ((H, H), lambda i: (0, 0))

    def wspec(shape):
        return pl.BlockSpec(shape, lambda i: (0, 0))

    w1_0 = blocks[0]["edge_mlp"]["W1"]
    x, xs, xr = pl.pallas_call(
        _node_enc_body,
        out_shape=[jax.ShapeDtypeStruct((n, H), _F32)] * 3,
    )(node_features, *_w(params["node_enc"]),
      w1_0[H:2 * H], w1_0[2 * H:3 * H])

    grid = (e // BE,)
    ein = edge_attr.shape[1]
    we = _w(params["edge_enc"])
    ee = pl.pallas_call(
        _edge_enc_body,
        grid=grid,
        in_specs=[
            pl.BlockSpec((BE, ein), lambda i: (i, 0)),
            wspec((ein, H)), wspec((1, H)), wspec((H, H)), wspec((1, H)),
            wspec((1, H)), wspec((1, H)),
        ],
        out_specs=pl.BlockSpec((BE, H), lambda i: (i, 0)),
        out_shape=jax.ShapeDtypeStruct((e, H), _F32),
    )(edge_attr, *we)

    gather = _make_gather(n, e)
    scatter = _make_scatter(n, e)

    def edge_update(ecur, g, emlp):
        w1, b1, w2, b2, ls, lb = _w(emlp)
        return pl.pallas_call(
            _edge_upd_body,
            grid=grid,
            in_specs=[
                pl.BlockSpec((BE, H), lambda i: (i, 0)),
                pl.BlockSpec((BE, H), lambda i: (i, 0)),
                wspec((H, H)), wspec((1, H)), wspec((H, H)), wspec((1, H)),
                wspec((1, H)), wspec((1, H)),
            ],
            out_specs=pl.BlockSpec((BE, H), lambda i: (i, 0)),
            out_shape=jax.ShapeDtypeStruct((e, H), _F32),
        )(ecur, g, w1[:H], b1, w2, b2, ls, lb)

    ecur = ee
    for i, blk in enumerate(blocks):
        g = gather(xs, xr, senders, receivers)
        ecur = edge_update(ecur, g, blk["edge_mlp"])
        agg2 = scatter(ecur, receivers)
        nm = blk["node_mlp"]
        w1, b1, w2, b2, ls, lb = _w(nm)
        if i + 1 < nblk:
            w1n = blocks[i + 1]["edge_mlp"]["W1"]
            x, xs, xr = pl.pallas_call(
                _node_upd_body,
                out_shape=[jax.ShapeDtypeStruct((n, H), _F32)] * 3,
            )(x, agg2, w1[:H], w1[H:2 * H], b1, w2, b2, ls, lb,
              w1n[H:2 * H], w1n[2 * H:3 * H])
        else:
            dec = params["decoder"]
            out = pl.pallas_call(
                _node_dec_body,
                out_shape=jax.ShapeDtypeStruct((n, dec["W2"].shape[1]), _F32),
            )(x, agg2, w1[:H], w1[H:2 * H], b1, w2, b2, ls, lb,
              dec["W1"], _row(dec["b1"]), dec["W2"], _row(dec["b2"]))
    return out


# speedup vs baseline: 5.1916x; 1.0397x over previous
import functools

import jax
import jax.numpy as jnp
from jax import lax
from jax.experimental import pallas as pl
from jax.experimental.pallas import tpu as pltpu
from jax.experimental.pallas import tpu_sc as plsc

H = 128
NC = 2
NS = 16
NW = NC * NS
CH = 128
BE = 8000
NSPLIT = 2

_F32 = jnp.float32


def _ln(y, scale, bias):
    mu = jnp.mean(y, axis=-1, keepdims=True)
    yc = y - mu
    var = jnp.mean(yc * yc, axis=-1, keepdims=True)
    return yc * lax.rsqrt(var + 1e-5) * scale + bias


def _dot(a, b):
    return jnp.dot(a, b, preferred_element_type=_F32)


def _node_enc_body(x_ref, w1, b1, w2, b2, ls, lb, w1s, w1r, xo, xso, xro):
    h = jnp.maximum(_dot(x_ref[:], w1[:]) + b1[:], 0.0)
    x = _ln(_dot(h, w2[:]) + b2[:], ls[:], lb[:])
    xo[:] = x
    xso[:] = _dot(x, w1s[:])
    xro[:] = _dot(x, w1r[:])


def _edge_enc_body(a_ref, w1, b1, w2, b2, ls, lb, eo):
    h = jnp.maximum(_dot(a_ref[:], w1[:]) + b1[:], 0.0)
    eo[:] = _ln(_dot(h, w2[:]) + b2[:], ls[:], lb[:])


def _edge_upd_body(e_ref, g_ref, w1e, b1, w2, b2, ls, lb, eo):
    h = jnp.maximum(_dot(e_ref[:], w1e[:]) + g_ref[:] + b1[:], 0.0)
    eo[:] = _ln(_dot(h, w2[:]) + b2[:], ls[:], lb[:]) + e_ref[:]


def _sum_aggs(agg_refs):
    agg = agg_refs[0][0] + agg_refs[0][1]
    for a in agg_refs[1:]:
        agg = agg + a[0] + a[1]
    return agg


def _make_node_upd_body(nsplit):
    def body(x_ref, *rest):
        agg_refs = rest[:nsplit]
        (w1x, w1a, b1, w2, b2, ls, lb, w1s, w1r, xo, xso, xro) = rest[nsplit:]
        agg = _sum_aggs(agg_refs)
        h = jnp.maximum(
            _dot(x_ref[:], w1x[:]) + _dot(agg, w1a[:]) + b1[:], 0.0)
        x = _ln(_dot(h, w2[:]) + b2[:], ls[:], lb[:]) + x_ref[:]
        xo[:] = x
        xso[:] = _dot(x, w1s[:])
        xro[:] = _dot(x, w1r[:])
    return body


def _make_node_dec_body(nsplit):
    def body(x_ref, *rest):
        agg_refs = rest[:nsplit]
        (w1x, w1a, b1, w2, b2, ls, lb, wd1, bd1, wd2, bd2, oo) = rest[nsplit:]
        agg = _sum_aggs(agg_refs)
        h = jnp.maximum(
            _dot(x_ref[:], w1x[:]) + _dot(agg, w1a[:]) + b1[:], 0.0)
        x = _ln(_dot(h, w2[:]) + b2[:], ls[:], lb[:]) + x_ref[:]
        hd = jnp.maximum(_dot(x, wd1[:]) + bd1[:], 0.0)
        oo[:] = _dot(hd, wd2[:]) + bd2[:]
    return body


def _row(v):
    return v.reshape(1, -1)


@functools.lru_cache(maxsize=None)
def _make_gather(n, e):
    nchunk = e // CH
    mesh = plsc.VectorSubcoreMesh(core_axis_name="c", subcore_axis_name="s")

    @functools.partial(
        pl.kernel,
        name="sc_gather_add",
        out_type=jax.ShapeDtypeStruct((e, H), _F32),
        mesh=mesh,
        scratch_types=[
            pltpu.VMEM((2, CH), jnp.int32),
            pltpu.VMEM((2, CH), jnp.int32),
            pltpu.VMEM((2, CH, H), _F32),
            pltpu.VMEM((2, CH, H), _F32),
            pltpu.SemaphoreType.DMA,
            pltpu.SemaphoreType.DMA,
            pltpu.SemaphoreType.DMA,
            pltpu.SemaphoreType.DMA,
        ],
    )
    def gather(xs_hbm, xr_hbm, s_hbm, r_hbm, out_hbm,
               sidx, ridx, bufa, bufb, semg0, semg1, semw0, semw1):
        wid = lax.axis_index("s") * NC + lax.axis_index("c")
        nloc = (nchunk - wid + NW - 1) // NW
        semg = (semg0, semg1)
        semw = (semw0, semw1)

        def fetch(k, b):
            base = (wid + k * NW) * CH
            pltpu.sync_copy(s_hbm.at[pl.ds(base, CH)], sidx.at[b])
            pltpu.sync_copy(r_hbm.at[pl.ds(base, CH)], ridx.at[b])
            pltpu.async_copy(xs_hbm.at[sidx.at[b]], bufa.at[b], semg[b])
            pltpu.async_copy(xr_hbm.at[ridx.at[b]], bufb.at[b], semg[b])

        fetch(0, 0)

        def group(g, carry):
            for b in range(2):
                k = 2 * g + b

                @pl.when(k < nloc)
                def _():
                    nb = 1 - b

                    @pl.when(k + 1 < nloc)
                    def _():
                        @pl.when(k >= 1)
                        def _():
                            pltpu.make_async_copy(
                                bufa.at[nb], out_hbm.at[pl.ds(0, CH)],
                                semw[nb]).wait()

                        fetch(k + 1, nb)

                    pltpu.make_async_copy(
                        xs_hbm.at[sidx.at[b]], bufa.at[b], semg[b]).wait()
                    pltpu.make_async_copy(
                        xr_hbm.at[ridx.at[b]], bufb.at[b], semg[b]).wait()

                    def addrow(rr, c2):
                        for j in range(H // 16):
                            sl = pl.ds(j * 16, 16)
                            bufa[b, rr, sl] = bufa[b, rr, sl] + bufb[b, rr, sl]
                        return c2

                    lax.fori_loop(0, CH, addrow, 0)
                    base = (wid + k * NW) * CH
                    pltpu.async_copy(bufa.at[b], out_hbm.at[pl.ds(base, CH)],
                                     semw[b])
            return carry

        lax.fori_loop(0, (nloc + 1) // 2, group, 0)
        for b in range(2):
            pltpu.make_async_copy(
                bufa.at[b], out_hbm.at[pl.ds(0, CH)], semw[b]).wait()

    return gather


@functools.lru_cache(maxsize=None)
def _make_scatter(n, e):
    nchunk = e // CH
    stripe = (n // NS) // 8 * 8
    tail = n - stripe * NS
    zrows = 104
    nz = stripe // zrows
    mesh = plsc.VectorSubcoreMesh(core_axis_name="c", subcore_axis_name="s")

    @functools.partial(
        pl.kernel,
        name="sc_segment_sum",
        out_type=jax.ShapeDtypeStruct((NC, n, H), _F32),
        mesh=mesh,
        scratch_types=[
            pltpu.VMEM((2, CH), jnp.int32),
            pltpu.VMEM((2, CH, H), _F32),
            pltpu.VMEM((zrows, H), _F32),
            pltpu.VMEM_SHARED((n, H), _F32),
            pltpu.SemaphoreType.DMA,
            pltpu.SemaphoreType.DMA,
        ],
    )
    def scatter(vals_hbm, r_hbm, out_hbm, idxv, rowsv, zbuf, acc,
                semr0, semr1):
        cid = lax.axis_index("c")
        sid = lax.axis_index("s")
        wid = sid * NC + cid
        semr = (semr0, semr1)

        def zrow(i, c2):
            for j in range(H // 16):
                zbuf[i, pl.ds(j * 16, 16)] = jnp.zeros((16,), _F32)
            return c2

        lax.fori_loop(0, zrows, zrow, 0)

        def zcp(k, c2):
            pltpu.sync_copy(zbuf, acc.at[pl.ds(sid * stripe + k * zrows, zrows)])
            return c2

        lax.fori_loop(0, nz, zcp, 0)
        if tail:
            @pl.when(sid == NS - 1)
            def _():
                pltpu.sync_copy(zbuf.at[pl.ds(0, tail)],
                                acc.at[pl.ds(NS * stripe, tail)])
        plsc.subcore_barrier()

        nloc = (nchunk - wid + NW - 1) // NW

        def fetch(k, b):
            base = (wid + k * NW) * CH
            pltpu.async_copy(r_hbm.at[pl.ds(base, CH)], idxv.at[b], semr[b])
            pltpu.async_copy(vals_hbm.at[pl.ds(base, CH)], rowsv.at[b],
                             semr[b])

        fetch(0, 0)

        def group(g, carry):
            for b in range(2):
                k = 2 * g + b

                @pl.when(k < nloc)
                def _():
                    @pl.when(k + 1 < nloc)
                    def _():
                        fetch(k + 1, 1 - b)

                    pltpu.make_async_copy(
                        r_hbm.at[pl.ds(0, CH)], idxv.at[b], semr[b]).wait()
                    pltpu.make_async_copy(
                        vals_hbm.at[pl.ds(0, CH)], rowsv.at[b],
                        semr[b]).wait()
                    pltpu.sync_copy(rowsv.at[b], acc.at[idxv.at[b]], add=True)
            return carry

        lax.fori_loop(0, (nloc + 1) // 2, group, 0)
        plsc.subcore_barrier()
        pltpu.sync_copy(
            acc.at[pl.ds(sid * stripe, stripe)],
            out_hbm.at[cid, pl.ds(sid * stripe, stripe)])
        if tail:
            @pl.when(sid == NS - 1)
            def _():
                pltpu.sync_copy(
                    acc.at[pl.ds(NS * stripe, tail)],
                    out_hbm.at[cid, pl.ds(NS * stripe, tail)])

    return scatter


def _w(p):
    return (p["W1"], _row(p["b1"]), p["W2"], _row(p["b2"]),
            _row(p["ln_scale"]), _row(p["ln_bias"]))


def kernel(node_features, edge_attr, senders, receivers, params):
    n = node_features.shape[0]
    e = edge_attr.shape[0]
    blocks = params["blocks"]
    nblk = len(blocks)

    hh = ((H, H), lambda i: (0, 0))

    def wspec(shape):
        return pl.BlockSpec(shape, lambda i: (0, 0))

    w1_0 = blocks[0]["edge_mlp"]["W1"]
    x, xs, xr = pl.pallas_call(
        _node_enc_body,
        out_shape=[jax.ShapeDtypeStruct((n, H), _F32)] * 3,
    )(node_features, *_w(params["node_enc"]),
      w1_0[H:2 * H], w1_0[2 * H:3 * H])

    e2 = e // NSPLIT
    grid = (e2 // BE,)
    ein = edge_attr.shape[1]
    we = _w(params["edge_enc"])
    snd = [senders[j * e2:(j + 1) * e2] for j in range(NSPLIT)]
    rcv = [receivers[j * e2:(j + 1) * e2] for j in range(NSPLIT)]

    def edge_encode(attr):
        return pl.pallas_call(
            _edge_enc_body,
            grid=grid,
            in_specs=[
                pl.BlockSpec((BE, ein), lambda i: (i, 0)),
                wspec((ein, H)), wspec((1, H)), wspec((H, H)), wspec((1, H)),
                wspec((1, H)), wspec((1, H)),
            ],
            out_specs=pl.BlockSpec((BE, H), lambda i: (i, 0)),
            out_shape=jax.ShapeDtypeStruct((e2, H), _F32),
        )(attr, *we)

    ecur = [edge_encode(edge_attr[j * e2:(j + 1) * e2]) for j in range(NSPLIT)]

    gather = _make_gather(n, e2)
    scatter = _make_scatter(n, e2)

    def edge_update(ej, g, emlp):
        w1, b1, w2, b2, ls, lb = _w(emlp)
        return pl.pallas_call(
            _edge_upd_body,
            grid=grid,
            in_specs=[
                pl.BlockSpec((BE, H), lambda i: (i, 0)),
                pl.BlockSpec((BE, H), lambda i: (i, 0)),
                wspec((H, H)), wspec((1, H)), wspec((H, H)), wspec((1, H)),
                wspec((1, H)), wspec((1, H)),
            ],
            out_specs=pl.BlockSpec((BE, H), lambda i: (i, 0)),
            out_shape=jax.ShapeDtypeStruct((e2, H), _F32),
        )(ej, g, w1[:H], b1, w2, b2, ls, lb)

    for i, blk in enumerate(blocks):
        gs = [gather(xs, xr, snd[j], rcv[j]) for j in range(NSPLIT)]
        ecur = [edge_update(ecur[j], gs[j], blk["edge_mlp"])
                for j in range(NSPLIT)]
        aggs = [scatter(ecur[j], rcv[j]) for j in range(NSPLIT)]
        nm = blk["node_mlp"]
        w1, b1, w2, b2, ls, lb = _w(nm)
        if i + 1 < nblk:
            w1n = blocks[i + 1]["edge_mlp"]["W1"]
            x, xs, xr = pl.pallas_call(
                _make_node_upd_body(NSPLIT),
                out_shape=[jax.ShapeDtypeStruct((n, H), _F32)] * 3,
            )(x, *aggs, w1[:H], w1[H:2 * H], b1, w2, b2, ls, lb,
              w1n[H:2 * H], w1n[2 * H:3 * H])
        else:
            dec = params["decoder"]
            out = pl.pallas_call(
                _make_node_dec_body(NSPLIT),
                out_shape=jax.ShapeDtypeStruct((n, dec["W2"].shape[1]), _F32),
            )(x, *aggs, w1[:H], w1[H:2 * H], b1, w2, b2, ls, lb,
              dec["W1"], _row(dec["b1"]), dec["W2"], _row(dec["b2"]))
    return out


# speedup vs baseline: 5.4300x; 1.0459x over previous
import functools

import jax
import jax.numpy as jnp
from jax import lax
from jax.experimental import pallas as pl
from jax.experimental.pallas import tpu as pltpu
from jax.experimental.pallas import tpu_sc as plsc

H = 128
NC = 2
NS = 16
NW = NC * NS
CH = 128
BE = 8000
NSPLIT = 2

_F32 = jnp.float32


def _ln(y, scale, bias):
    mu = jnp.mean(y, axis=-1, keepdims=True)
    yc = y - mu
    var = jnp.mean(yc * yc, axis=-1, keepdims=True)
    return yc * lax.rsqrt(var + 1e-5) * scale + bias


def _dot(a, b):
    return jnp.dot(a, b, preferred_element_type=_F32)


def _node_enc_body(x_ref, w1, b1, w2, b2, ls, lb, w1s, w1r, xo, xso, xro):
    h = jnp.maximum(_dot(x_ref[:], w1[:]) + b1[:], 0.0)
    x = _ln(_dot(h, w2[:]) + b2[:], ls[:], lb[:])
    xo[:] = x
    xso[:] = _dot(x, w1s[:])
    xro[:] = _dot(x, w1r[:])


def _edge_enc_body(a_ref, w1, b1, w2, b2, ls, lb, eo):
    h = jnp.maximum(_dot(a_ref[:], w1[:]) + b1[:], 0.0)
    eo[:] = _ln(_dot(h, w2[:]) + b2[:], ls[:], lb[:])


def _edge_upd_body(e_ref, g_ref, w1e, b1, w2, b2, ls, lb, eo):
    h = jnp.maximum(_dot(e_ref[:], w1e[:]) + g_ref[:] + b1[:], 0.0)
    eo[:] = _ln(_dot(h, w2[:]) + b2[:], ls[:], lb[:]) + e_ref[:]


def _sum_aggs(agg_refs):
    agg = agg_refs[0][0] + agg_refs[0][1]
    for a in agg_refs[1:]:
        agg = agg + a[0] + a[1]
    return agg


def _make_node_upd_body(nsplit):
    def body(x_ref, *rest):
        agg_refs = rest[:nsplit]
        (w1x, w1a, b1, w2, b2, ls, lb, w1s, w1r, xo, xso, xro) = rest[nsplit:]
        agg = _sum_aggs(agg_refs)
        h = jnp.maximum(
            _dot(x_ref[:], w1x[:]) + _dot(agg, w1a[:]) + b1[:], 0.0)
        x = _ln(_dot(h, w2[:]) + b2[:], ls[:], lb[:]) + x_ref[:]
        xo[:] = x
        xso[:] = _dot(x, w1s[:])
        xro[:] = _dot(x, w1r[:])
    return body


def _make_node_dec_body(nsplit):
    def body(x_ref, *rest):
        agg_refs = rest[:nsplit]
        (w1x, w1a, b1, w2, b2, ls, lb, wd1, bd1, wd2, bd2, oo) = rest[nsplit:]
        agg = _sum_aggs(agg_refs)
        h = jnp.maximum(
            _dot(x_ref[:], w1x[:]) + _dot(agg, w1a[:]) + b1[:], 0.0)
        x = _ln(_dot(h, w2[:]) + b2[:], ls[:], lb[:]) + x_ref[:]
        hd = jnp.maximum(_dot(x, wd1[:]) + bd1[:], 0.0)
        oo[:] = _dot(hd, wd2[:]) + bd2[:]
    return body


def _row(v):
    return v.reshape(1, -1)


@functools.lru_cache(maxsize=None)
def _make_gather(n, e):
    nchunk = e // CH
    mesh = plsc.VectorSubcoreMesh(core_axis_name="c", subcore_axis_name="s")

    @functools.partial(
        pl.kernel,
        name="sc_gather_add",
        out_type=jax.ShapeDtypeStruct((e, H), _F32),
        mesh=mesh,
        scratch_types=[
            pltpu.VMEM((3, CH), jnp.int32),
            pltpu.VMEM((3, CH), jnp.int32),
            pltpu.VMEM((3, CH, H), _F32),
            pltpu.VMEM((3, CH, H), _F32),
            pltpu.SemaphoreType.DMA,
            pltpu.SemaphoreType.DMA,
            pltpu.SemaphoreType.DMA,
            pltpu.SemaphoreType.DMA,
            pltpu.SemaphoreType.DMA,
            pltpu.SemaphoreType.DMA,
            pltpu.SemaphoreType.DMA,
            pltpu.SemaphoreType.DMA,
            pltpu.SemaphoreType.DMA,
        ],
    )
    def gather(xs_hbm, xr_hbm, s_hbm, r_hbm, out_hbm, sidx, ridx, bufa, bufb,
               semi0, semi1, semi2, semg0, semg1, semg2, semw0, semw1, semw2):
        wid = lax.axis_index("s") * NC + lax.axis_index("c")
        nloc = (nchunk - wid + NW - 1) // NW
        semi = (semi0, semi1, semi2)
        semg = (semg0, semg1, semg2)
        semw = (semw0, semw1, semw2)

        def fetch_idx(k, b):
            base = (wid + k * NW) * CH
            pltpu.async_copy(s_hbm.at[pl.ds(base, CH)], sidx.at[b], semi[b])
            pltpu.async_copy(r_hbm.at[pl.ds(base, CH)], ridx.at[b], semi[b])

        def fire_rows(b):
            pltpu.make_async_copy(
                s_hbm.at[pl.ds(0, CH)], sidx.at[b], semi[b]).wait()
            pltpu.make_async_copy(
                r_hbm.at[pl.ds(0, CH)], ridx.at[b], semi[b]).wait()
            pltpu.async_copy(xs_hbm.at[sidx.at[b]], bufa.at[b], semg[b])
            pltpu.async_copy(xr_hbm.at[ridx.at[b]], bufb.at[b], semg[b])

        fetch_idx(0, 0)
        fetch_idx(1, 1)
        fire_rows(0)

        def group(g, carry):
            for b in range(3):
                k = 3 * g + b

                @pl.when(k < nloc)
                def _():
                    b1 = (b + 1) % 3
                    b2 = (b + 2) % 3

                    @pl.when(k + 2 < nloc)
                    def _():
                        fetch_idx(k + 2, b2)

                    @pl.when(k + 1 < nloc)
                    def _():
                        @pl.when(k >= 2)
                        def _():
                            pltpu.make_async_copy(
                                bufa.at[b1], out_hbm.at[pl.ds(0, CH)],
                                semw[b1]).wait()

                        fire_rows(b1)

                    pltpu.make_async_copy(
                        xs_hbm.at[sidx.at[b]], bufa.at[b], semg[b]).wait()
                    pltpu.make_async_copy(
                        xr_hbm.at[ridx.at[b]], bufb.at[b], semg[b]).wait()

                    def addrow(rr, c2):
                        for j in range(H // 16):
                            sl = pl.ds(j * 16, 16)
                            bufa[b, rr, sl] = bufa[b, rr, sl] + bufb[b, rr, sl]
                        return c2

                    lax.fori_loop(0, CH, addrow, 0)
                    base = (wid + k * NW) * CH
                    pltpu.async_copy(bufa.at[b], out_hbm.at[pl.ds(base, CH)],
                                     semw[b])
            return carry

        lax.fori_loop(0, (nloc + 2) // 3, group, 0)
        for b in range(3):
            pltpu.make_async_copy(
                bufa.at[b], out_hbm.at[pl.ds(0, CH)], semw[b]).wait()

    return gather


@functools.lru_cache(maxsize=None)
def _make_scatter(n, e):
    nchunk = e // CH
    stripe = (n // NS) // 8 * 8
    tail = n - stripe * NS
    zrows = 104
    nz = stripe // zrows
    mesh = plsc.VectorSubcoreMesh(core_axis_name="c", subcore_axis_name="s")

    @functools.partial(
        pl.kernel,
        name="sc_segment_sum",
        out_type=jax.ShapeDtypeStruct((NC, n, H), _F32),
        mesh=mesh,
        scratch_types=[
            pltpu.VMEM((2, CH), jnp.int32),
            pltpu.VMEM((2, CH, H), _F32),
            pltpu.VMEM((zrows, H), _F32),
            pltpu.VMEM_SHARED((n, H), _F32),
            pltpu.SemaphoreType.DMA,
            pltpu.SemaphoreType.DMA,
        ],
    )
    def scatter(vals_hbm, r_hbm, out_hbm, idxv, rowsv, zbuf, acc,
                semr0, semr1):
        cid = lax.axis_index("c")
        sid = lax.axis_index("s")
        wid = sid * NC + cid
        semr = (semr0, semr1)

        def zrow(i, c2):
            for j in range(H // 16):
                zbuf[i, pl.ds(j * 16, 16)] = jnp.zeros((16,), _F32)
            return c2

        lax.fori_loop(0, zrows, zrow, 0)

        def zcp(k, c2):
            pltpu.sync_copy(zbuf, acc.at[pl.ds(sid * stripe + k * zrows, zrows)])
            return c2

        lax.fori_loop(0, nz, zcp, 0)
        if tail:
            @pl.when(sid == NS - 1)
            def _():
                pltpu.sync_copy(zbuf.at[pl.ds(0, tail)],
                                acc.at[pl.ds(NS * stripe, tail)])
        plsc.subcore_barrier()

        nloc = (nchunk - wid + NW - 1) // NW

        def fetch(k, b):
            base = (wid + k * NW) * CH
            pltpu.async_copy(r_hbm.at[pl.ds(base, CH)], idxv.at[b], semr[b])
            pltpu.async_copy(vals_hbm.at[pl.ds(base, CH)], rowsv.at[b],
                             semr[b])

        fetch(0, 0)

        def group(g, carry):
            for b in range(2):
                k = 2 * g + b

                @pl.when(k < nloc)
                def _():
                    @pl.when(k + 1 < nloc)
                    def _():
                        fetch(k + 1, 1 - b)

                    pltpu.make_async_copy(
                        r_hbm.at[pl.ds(0, CH)], idxv.at[b], semr[b]).wait()
                    pltpu.make_async_copy(
                        vals_hbm.at[pl.ds(0, CH)], rowsv.at[b],
                        semr[b]).wait()
                    pltpu.sync_copy(rowsv.at[b], acc.at[idxv.at[b]], add=True)
            return carry

        lax.fori_loop(0, (nloc + 1) // 2, group, 0)
        plsc.subcore_barrier()
        pltpu.sync_copy(
            acc.at[pl.ds(sid * stripe, stripe)],
            out_hbm.at[cid, pl.ds(sid * stripe, stripe)])
        if tail:
            @pl.when(sid == NS - 1)
            def _():
                pltpu.sync_copy(
                    acc.at[pl.ds(NS * stripe, tail)],
                    out_hbm.at[cid, pl.ds(NS * stripe, tail)])

    return scatter


def _w(p):
    return (p["W1"], _row(p["b1"]), p["W2"], _row(p["b2"]),
            _row(p["ln_scale"]), _row(p["ln_bias"]))


def kernel(node_features, edge_attr, senders, receivers, params):
    n = node_features.shape[0]
    e = edge_attr.shape[0]
    blocks = params["blocks"]
    nblk = len(blocks)

    hh = ((H, H), lambda i: (0, 0))

    def wspec(shape):
        return pl.BlockSpec(shape, lambda i: (0, 0))

    w1_0 = blocks[0]["edge_mlp"]["W1"]
    x, xs, xr = pl.pallas_call(
        _node_enc_body,
        out_shape=[jax.ShapeDtypeStruct((n, H), _F32)] * 3,
    )(node_features, *_w(params["node_enc"]),
      w1_0[H:2 * H], w1_0[2 * H:3 * H])

    e2 = e // NSPLIT
    grid = (e2 // BE,)
    ein = edge_attr.shape[1]
    we = _w(params["edge_enc"])
    snd = [senders[j * e2:(j + 1) * e2] for j in range(NSPLIT)]
    rcv = [receivers[j * e2:(j + 1) * e2] for j in range(NSPLIT)]

    def edge_encode(attr):
        return pl.pallas_call(
            _edge_enc_body,
            grid=grid,
            in_specs=[
                pl.BlockSpec((BE, ein), lambda i: (i, 0)),
                wspec((ein, H)), wspec((1, H)), wspec((H, H)), wspec((1, H)),
                wspec((1, H)), wspec((1, H)),
            ],
            out_specs=pl.BlockSpec((BE, H), lambda i: (i, 0)),
            out_shape=jax.ShapeDtypeStruct((e2, H), _F32),
        )(attr, *we)

    ecur = [edge_encode(edge_attr[j * e2:(j + 1) * e2]) for j in range(NSPLIT)]

    gather = _make_gather(n, e2)
    scatter = _make_scatter(n, e2)

    def edge_update(ej, g, emlp):
        w1, b1, w2, b2, ls, lb = _w(emlp)
        return pl.pallas_call(
            _edge_upd_body,
            grid=grid,
            in_specs=[
                pl.BlockSpec((BE, H), lambda i: (i, 0)),
                pl.BlockSpec((BE, H), lambda i: (i, 0)),
                wspec((H, H)), wspec((1, H)), wspec((H, H)), wspec((1, H)),
                wspec((1, H)), wspec((1, H)),
            ],
            out_specs=pl.BlockSpec((BE, H), lambda i: (i, 0)),
            out_shape=jax.ShapeDtypeStruct((e2, H), _F32),
        )(ej, g, w1[:H], b1, w2, b2, ls, lb)

    for i, blk in enumerate(blocks):
        gs = [gather(xs, xr, snd[j], rcv[j]) for j in range(NSPLIT)]
        ecur = [edge_update(ecur[j], gs[j], blk["edge_mlp"])
                for j in range(NSPLIT)]
        aggs = [scatter(ecur[j], rcv[j]) for j in range(NSPLIT)]
        nm = blk["node_mlp"]
        w1, b1, w2, b2, ls, lb = _w(nm)
        if i + 1 < nblk:
            w1n = blocks[i + 1]["edge_mlp"]["W1"]
            x, xs, xr = pl.pallas_call(
                _make_node_upd_body(NSPLIT),
                out_shape=[jax.ShapeDtypeStruct((n, H), _F32)] * 3,
            )(x, *aggs, w1[:H], w1[H:2 * H], b1, w2, b2, ls, lb,
              w1n[H:2 * H], w1n[2 * H:3 * H])
        else:
            dec = params["decoder"]
            out = pl.pallas_call(
                _make_node_dec_body(NSPLIT),
                out_shape=jax.ShapeDtypeStruct((n, dec["W2"].shape[1]), _F32),
            )(x, *aggs, w1[:H], w1[H:2 * H], b1, w2, b2, ls, lb,
              dec["W1"], _row(dec["b1"]), dec["W2"], _row(dec["b2"]))
    return out


# speedup vs baseline: 5.4304x; 1.0001x over previous
import functools

import jax
import jax.numpy as jnp
from jax import lax
from jax.experimental import pallas as pl
from jax.experimental.pallas import tpu as pltpu
from jax.experimental.pallas import tpu_sc as plsc

H = 128
NC = 2
NS = 16
NW = NC * NS
CH = 128
BE = 8000
NSPLIT = 2

_F32 = jnp.float32


def _ln(y, scale, bias):
    mu = jnp.mean(y, axis=-1, keepdims=True)
    yc = y - mu
    var = jnp.mean(yc * yc, axis=-1, keepdims=True)
    return yc * lax.rsqrt(var + 1e-5) * scale + bias


def _dot(a, b):
    return jnp.dot(a, b, preferred_element_type=_F32)


def _node_enc_body(x_ref, w1, b1, w2, b2, ls, lb, w1s, w1r, xo, xso, xro):
    h = jnp.maximum(_dot(x_ref[:], w1[:]) + b1[:], 0.0)
    x = _ln(_dot(h, w2[:]) + b2[:], ls[:], lb[:])
    xo[:] = x
    xso[:] = _dot(x, w1s[:])
    xro[:] = _dot(x, w1r[:])


def _edge_enc_body(a_ref, w1, b1, w2, b2, ls, lb, eo):
    h = jnp.maximum(_dot(a_ref[:], w1[:]) + b1[:], 0.0)
    eo[:] = _ln(_dot(h, w2[:]) + b2[:], ls[:], lb[:])


def _edge_upd_body(e_ref, g_ref, w1e, b1, w2, b2, ls, lb, eo):
    h = jnp.maximum(_dot(e_ref[:], w1e[:]) + g_ref[:] + b1[:], 0.0)
    eo[:] = _ln(_dot(h, w2[:]) + b2[:], ls[:], lb[:]) + e_ref[:]


def _sum_aggs(agg_refs):
    agg = agg_refs[0][0] + agg_refs[0][1]
    for a in agg_refs[1:]:
        agg = agg + a[0] + a[1]
    return agg


def _make_node_upd_body(nsplit):
    def body(x_ref, *rest):
        agg_refs = rest[:nsplit]
        (w1x, w1a, b1, w2, b2, ls, lb, w1s, w1r, xo, xso, xro) = rest[nsplit:]
        agg = _sum_aggs(agg_refs)
        h = jnp.maximum(
            _dot(x_ref[:], w1x[:]) + _dot(agg, w1a[:]) + b1[:], 0.0)
        x = _ln(_dot(h, w2[:]) + b2[:], ls[:], lb[:]) + x_ref[:]
        xo[:] = x
        xso[:] = _dot(x, w1s[:])
        xro[:] = _dot(x, w1r[:])
    return body


def _make_node_dec_body(nsplit):
    def body(x_ref, *rest):
        agg_refs = rest[:nsplit]
        (w1x, w1a, b1, w2, b2, ls, lb, wd1, bd1, wd2, bd2, oo) = rest[nsplit:]
        agg = _sum_aggs(agg_refs)
        h = jnp.maximum(
            _dot(x_ref[:], w1x[:]) + _dot(agg, w1a[:]) + b1[:], 0.0)
        x = _ln(_dot(h, w2[:]) + b2[:], ls[:], lb[:]) + x_ref[:]
        hd = jnp.maximum(_dot(x, wd1[:]) + bd1[:], 0.0)
        oo[:] = _dot(hd, wd2[:]) + bd2[:]
    return body


def _row(v):
    return v.reshape(1, -1)


@functools.lru_cache(maxsize=None)
def _make_gather(n, e):
    nchunk = e // CH
    mesh = plsc.VectorSubcoreMesh(core_axis_name="c", subcore_axis_name="s")

    @functools.partial(
        pl.kernel,
        name="sc_gather_add",
        out_type=jax.ShapeDtypeStruct((e, H), _F32),
        mesh=mesh,
        scratch_types=[
            pltpu.VMEM((3, CH), jnp.int32),
            pltpu.VMEM((3, CH), jnp.int32),
            pltpu.VMEM((3, CH, H), _F32),
            pltpu.VMEM((3, CH, H), _F32),
            pltpu.SemaphoreType.DMA,
            pltpu.SemaphoreType.DMA,
            pltpu.SemaphoreType.DMA,
            pltpu.SemaphoreType.DMA,
            pltpu.SemaphoreType.DMA,
            pltpu.SemaphoreType.DMA,
            pltpu.SemaphoreType.DMA,
            pltpu.SemaphoreType.DMA,
            pltpu.SemaphoreType.DMA,
        ],
    )
    def gather(xs_hbm, xr_hbm, s_hbm, r_hbm, out_hbm, sidx, ridx, bufa, bufb,
               semi0, semi1, semi2, semg0, semg1, semg2, semw0, semw1, semw2):
        wid = lax.axis_index("s") * NC + lax.axis_index("c")
        nloc = (nchunk - wid + NW - 1) // NW
        semi = (semi0, semi1, semi2)
        semg = (semg0, semg1, semg2)
        semw = (semw0, semw1, semw2)

        def fetch_idx(k, b):
            base = (wid + k * NW) * CH
            pltpu.async_copy(s_hbm.at[pl.ds(base, CH)], sidx.at[b], semi[b])
            pltpu.async_copy(r_hbm.at[pl.ds(base, CH)], ridx.at[b], semi[b])

        def fire_rows(b):
            pltpu.make_async_copy(
                s_hbm.at[pl.ds(0, CH)], sidx.at[b], semi[b]).wait()
            pltpu.make_async_copy(
                r_hbm.at[pl.ds(0, CH)], ridx.at[b], semi[b]).wait()
            pltpu.async_copy(xs_hbm.at[sidx.at[b]], bufa.at[b], semg[b])
            pltpu.async_copy(xr_hbm.at[ridx.at[b]], bufb.at[b], semg[b])

        fetch_idx(0, 0)
        fetch_idx(1, 1)
        fire_rows(0)

        def group(g, carry):
            for b in range(3):
                k = 3 * g + b

                @pl.when(k < nloc)
                def _():
                    b1 = (b + 1) % 3
                    b2 = (b + 2) % 3

                    @pl.when(k + 2 < nloc)
                    def _():
                        fetch_idx(k + 2, b2)

                    @pl.when(k + 1 < nloc)
                    def _():
                        @pl.when(k >= 2)
                        def _():
                            pltpu.make_async_copy(
                                bufa.at[b1], out_hbm.at[pl.ds(0, CH)],
                                semw[b1]).wait()

                        fire_rows(b1)

                    pltpu.make_async_copy(
                        xs_hbm.at[sidx.at[b]], bufa.at[b], semg[b]).wait()
                    pltpu.make_async_copy(
                        xr_hbm.at[ridx.at[b]], bufb.at[b], semg[b]).wait()

                    def addrow(rr, c2):
                        for j in range(H // 16):
                            sl = pl.ds(j * 16, 16)
                            bufa[b, rr, sl] = bufa[b, rr, sl] + bufb[b, rr, sl]
                        return c2

                    lax.fori_loop(0, CH, addrow, 0)
                    base = (wid + k * NW) * CH
                    pltpu.async_copy(bufa.at[b], out_hbm.at[pl.ds(base, CH)],
                                     semw[b])
            return carry

        lax.fori_loop(0, (nloc + 2) // 3, group, 0)
        for b in range(3):
            pltpu.make_async_copy(
                bufa.at[b], out_hbm.at[pl.ds(0, CH)], semw[b]).wait()

    return gather


@functools.lru_cache(maxsize=None)
def _make_scatter(n, e):
    nchunk = e // CH
    stripe = (n // NS) // 8 * 8
    tail = n - stripe * NS
    zrows = 104
    nz = stripe // zrows
    mesh = plsc.VectorSubcoreMesh(core_axis_name="c", subcore_axis_name="s")

    @functools.partial(
        pl.kernel,
        name="sc_segment_sum",
        out_type=jax.ShapeDtypeStruct((NC, n, H), _F32),
        mesh=mesh,
        scratch_types=[
            pltpu.VMEM((2, CH), jnp.int32),
            pltpu.VMEM((2, CH, H), _F32),
            pltpu.VMEM((zrows, H), _F32),
            pltpu.VMEM_SHARED((n, H), _F32),
            pltpu.SemaphoreType.DMA,
            pltpu.SemaphoreType.DMA,
            pltpu.SemaphoreType.DMA,
        ],
    )
    def scatter(vals_hbm, r_hbm, out_hbm, idxv, rowsv, zbuf, acc,
                semr0, semr1, semz):
        cid = lax.axis_index("c")
        sid = lax.axis_index("s")
        wid = sid * NC + cid
        semr = (semr0, semr1)

        def zrow(i, c2):
            for j in range(H // 16):
                zbuf[i, pl.ds(j * 16, 16)] = jnp.zeros((16,), _F32)
            return c2

        lax.fori_loop(0, zrows, zrow, 0)

        for k in range(nz):
            pltpu.async_copy(
                zbuf, acc.at[pl.ds(sid * stripe + k * zrows, zrows)], semz)
        if tail:
            @pl.when(sid == NS - 1)
            def _():
                pltpu.async_copy(zbuf.at[pl.ds(0, tail)],
                                 acc.at[pl.ds(NS * stripe, tail)], semz)
        for k in range(nz):
            pltpu.make_async_copy(
                zbuf, acc.at[pl.ds(sid * stripe + k * zrows, zrows)],
                semz).wait()
        if tail:
            @pl.when(sid == NS - 1)
            def _():
                pltpu.make_async_copy(zbuf.at[pl.ds(0, tail)],
                                      acc.at[pl.ds(NS * stripe, tail)],
                                      semz).wait()
        plsc.subcore_barrier()

        nloc = (nchunk - wid + NW - 1) // NW

        def fetch(k, b):
            base = (wid + k * NW) * CH
            pltpu.async_copy(r_hbm.at[pl.ds(base, CH)], idxv.at[b], semr[b])
            pltpu.async_copy(vals_hbm.at[pl.ds(base, CH)], rowsv.at[b],
                             semr[b])

        fetch(0, 0)

        def group(g, carry):
            for b in range(2):
                k = 2 * g + b

                @pl.when(k < nloc)
                def _():
                    @pl.when(k + 1 < nloc)
                    def _():
                        fetch(k + 1, 1 - b)

                    pltpu.make_async_copy(
                        r_hbm.at[pl.ds(0, CH)], idxv.at[b], semr[b]).wait()
                    pltpu.make_async_copy(
                        vals_hbm.at[pl.ds(0, CH)], rowsv.at[b],
                        semr[b]).wait()
                    pltpu.sync_copy(rowsv.at[b], acc.at[idxv.at[b]], add=True)
            return carry

        lax.fori_loop(0, (nloc + 1) // 2, group, 0)
        plsc.subcore_barrier()
        pltpu.sync_copy(
            acc.at[pl.ds(sid * stripe, stripe)],
            out_hbm.at[cid, pl.ds(sid * stripe, stripe)])
        if tail:
            @pl.when(sid == NS - 1)
            def _():
                pltpu.sync_copy(
                    acc.at[pl.ds(NS * stripe, tail)],
                    out_hbm.at[cid, pl.ds(NS * stripe, tail)])

    return scatter


def _w(p):
    return (p["W1"], _row(p["b1"]), p["W2"], _row(p["b2"]),
            _row(p["ln_scale"]), _row(p["ln_bias"]))


def kernel(node_features, edge_attr, senders, receivers, params):
    n = node_features.shape[0]
    e = edge_attr.shape[0]
    blocks = params["blocks"]
    nblk = len(blocks)

    hh = ((H, H), lambda i: (0, 0))

    def wspec(shape):
        return pl.BlockSpec(shape, lambda i: (0, 0))

    w1_0 = blocks[0]["edge_mlp"]["W1"]
    x, xs, xr = pl.pallas_call(
        _node_enc_body,
        out_shape=[jax.ShapeDtypeStruct((n, H), _F32)] * 3,
    )(node_features, *_w(params["node_enc"]),
      w1_0[H:2 * H], w1_0[2 * H:3 * H])

    e2 = e // NSPLIT
    grid = (e2 // BE,)
    ein = edge_attr.shape[1]
    we = _w(params["edge_enc"])
    snd = [senders[j * e2:(j + 1) * e2] for j in range(NSPLIT)]
    rcv = [receivers[j * e2:(j + 1) * e2] for j in range(NSPLIT)]

    def edge_encode(attr):
        return pl.pallas_call(
            _edge_enc_body,
            grid=grid,
            in_specs=[
                pl.BlockSpec((BE, ein), lambda i: (i, 0)),
                wspec((ein, H)), wspec((1, H)), wspec((H, H)), wspec((1, H)),
                wspec((1, H)), wspec((1, H)),
            ],
            out_specs=pl.BlockSpec((BE, H), lambda i: (i, 0)),
            out_shape=jax.ShapeDtypeStruct((e2, H), _F32),
        )(attr, *we)

    ecur = [edge_encode(edge_attr[j * e2:(j + 1) * e2]) for j in range(NSPLIT)]

    gather = _make_gather(n, e2)
    scatter = _make_scatter(n, e2)

    def edge_update(ej, g, emlp):
        w1, b1, w2, b2, ls, lb = _w(emlp)
        return pl.pallas_call(
            _edge_upd_body,
            grid=grid,
            in_specs=[
                pl.BlockSpec((BE, H), lambda i: (i, 0)),
                pl.BlockSpec((BE, H), lambda i: (i, 0)),
                wspec((H, H)), wspec((1, H)), wspec((H, H)), wspec((1, H)),
                wspec((1, H)), wspec((1, H)),
            ],
            out_specs=pl.BlockSpec((BE, H), lambda i: (i, 0)),
            out_shape=jax.ShapeDtypeStruct((e2, H), _F32),
        )(ej, g, w1[:H], b1, w2, b2, ls, lb)

    for i, blk in enumerate(blocks):
        gs = [gather(xs, xr, snd[j], rcv[j]) for j in range(NSPLIT)]
        ecur = [edge_update(ecur[j], gs[j], blk["edge_mlp"])
                for j in range(NSPLIT)]
        aggs = [scatter(ecur[j], rcv[j]) for j in range(NSPLIT)]
        nm = blk["node_mlp"]
        w1, b1, w2, b2, ls, lb = _w(nm)
        if i + 1 < nblk:
            w1n = blocks[i + 1]["edge_mlp"]["W1"]
            x, xs, xr = pl.pallas_call(
                _make_node_upd_body(NSPLIT),
                out_shape=[jax.ShapeDtypeStruct((n, H), _F32)] * 3,
            )(x, *aggs, w1[:H], w1[H:2 * H], b1, w2, b2, ls, lb,
              w1n[H:2 * H], w1n[2 * H:3 * H])
        else:
            dec = params["decoder"]
            out = pl.pallas_call(
                _make_node_dec_body(NSPLIT),
                out_shape=jax.ShapeDtypeStruct((n, dec["W2"].shape[1]), _F32),
            )(x, *aggs, w1[:H], w1[H:2 * H], b1, w2, b2, ls, lb,
              dec["W1"], _row(dec["b1"]), dec["W2"], _row(dec["b2"]))
    return out
